# baseline (speedup 1.0000x reference)
import functools

import jax
import jax.numpy as jnp
import numpy as np
from jax import lax
from jax.experimental import pallas as pl
from jax.experimental.pallas import tpu as pltpu
from jax.experimental.pallas import tpu_sc as plsc

NC = 2
NS = 16
NW = NC * NS


def _unpack_perm(ncols):
    u = np.empty(ncols, np.int32)
    for k in range(ncols // 32):
        for i in range(16):
            u[32 * k + i] = 32 * k + 2 * i
            u[32 * k + 16 + i] = 32 * k + 2 * i + 1
    return u


def _tc_linear_bf16(X, Wt):
    N, D = X.shape
    OUT = Wt.shape[1]
    B = 1000

    def body(x_ref, w_ref, o_ref):
        acc = jnp.dot(x_ref[...], w_ref[...], preferred_element_type=jnp.float32)
        o_ref[...] = acc.astype(jnp.bfloat16)

    return pl.pallas_call(
        body,
        grid=(N // B,),
        in_specs=[pl.BlockSpec((B, D), lambda i: (i, 0)),
                  pl.BlockSpec((D, OUT), lambda i: (0, 0))],
        out_specs=pl.BlockSpec((B, OUT), lambda i: (i, 0)),
        out_shape=jax.ShapeDtypeStruct((N, OUT), jnp.bfloat16),
    )(X, Wt)


def _tc_mean_scale(partials, degE, OUT):
    _, M, DA = partials.shape
    B = 2000

    def body(p_ref, de_ref, o_ref):
        s = p_ref[0] + p_ref[1]
        cnt = s[:, OUT:OUT + 1]
        o_ref[...] = (s[:, :OUT] / jnp.maximum(cnt, 1.0)
                      * de_ref[...]).astype(jnp.bfloat16)

    return pl.pallas_call(
        body,
        grid=(M // B,),
        in_specs=[pl.BlockSpec((2, B, DA), lambda i: (0, i, 0)),
                  pl.BlockSpec((B, 1), lambda i: (i, 0))],
        out_specs=pl.BlockSpec((B, OUT), lambda i: (i, 0)),
        out_shape=jax.ShapeDtypeStruct((M, OUT), jnp.bfloat16),
    )(partials, degE)


def _tc_scale(partials, degV):
    _, N, OUT = partials.shape
    B = 2000

    def body(p_ref, dv_ref, o_ref):
        o_ref[...] = (p_ref[0] + p_ref[1]) * dv_ref[...]

    return pl.pallas_call(
        body,
        grid=(N // B,),
        in_specs=[pl.BlockSpec((2, B, OUT), lambda i: (0, i, 0)),
                  pl.BlockSpec((B, 1), lambda i: (i, 0))],
        out_specs=pl.BlockSpec((B, OUT), lambda i: (i, 0)),
        out_shape=jax.ShapeDtypeStruct((N, OUT), jnp.float32),
    )(partials, degV)


def _make_sc_stage(R, nnz, chunk, width, aug):
    D = 128
    P = nnz // NW
    nchunk = P // chunk
    rz = R // NS
    mesh = plsc.VectorSubcoreMesh(core_axis_name="c", subcore_axis_name="s")

    @functools.partial(
        pl.kernel,
        out_type=jax.ShapeDtypeStruct((NC, R, width), jnp.float32),
        mesh=mesh,
        compiler_params=pltpu.CompilerParams(use_tc_tiling_on_sc=False,
                                             needs_layout_passes=False),
        scratch_types=[
            pltpu.VMEM((chunk,), jnp.int32),
            pltpu.VMEM((chunk,), jnp.int32),
            pltpu.VMEM((chunk,), jnp.int32),
            pltpu.VMEM((chunk,), jnp.int32),
            pltpu.VMEM((chunk, D), jnp.bfloat16),
            pltpu.VMEM((chunk, D), jnp.bfloat16),
            pltpu.VMEM((chunk, width), jnp.float32),
            pltpu.VMEM((chunk, width), jnp.float32),
            pltpu.VMEM_SHARED((R, width), jnp.float32),
            pltpu.SemaphoreType.DMA,
            pltpu.SemaphoreType.DMA,
            pltpu.SemaphoreType.DMA,
            pltpu.SemaphoreType.DMA,
            pltpu.SemaphoreType.DMA,
            pltpu.SemaphoreType.DMA,
            pltpu.SemaphoreType.DMA,
            pltpu.SemaphoreType.DMA,
        ],
    )
    def stage(table, gidx, sidx, zeros, out, gb0, gb1, sb0, sb1, rb0, rb1,
              rf0, rf1, acc, gsem0, gsem1, glsem0, glsem1, isem0, isem1,
              ssem0, ssem1):
        cid = lax.axis_index("c")
        sid = lax.axis_index("s")
        wid = cid * NS + sid
        pltpu.sync_copy(zeros.at[pl.ds(sid * rz, rz)], acc.at[pl.ds(sid * rz, rz)])
        base0 = wid * P
        if aug:
            pad = jnp.where(lax.iota(jnp.int32, 16) == 0, 1.0, 0.0)
            for r in range(chunk):
                rf0[r, pl.ds(D, 16)] = pad
                rf1[r, pl.ds(D, 16)] = pad
        plsc.subcore_barrier()

        def start_gidx(j, gbuf, sem):
            pltpu.async_copy(gidx.at[pl.ds(base0 + j * chunk, chunk)],
                             gbuf, sem)

        def wait_gidx(j, gbuf, sem):
            pltpu.make_async_copy(gidx.at[pl.ds(base0 + j * chunk, chunk)],
                                  gbuf, sem).wait()

        def start_sidx(j, sbuf, sem):
            pltpu.async_copy(sidx.at[pl.ds(base0 + j * chunk, chunk)],
                             sbuf, sem)

        def wait_sidx(j, sbuf, sem):
            pltpu.make_async_copy(sidx.at[pl.ds(base0 + j * chunk, chunk)],
                                  sbuf, sem).wait()

        def convert(rbuf, fbuf):
            def quad(q, carry):
                for dr in range(4):
                    r = q * 4 + dr
                    for c in range(D // 32):
                        v = rbuf[r, pl.ds(32 * c, 32)]
                        a, b = plsc.unpack(
                            v, format=plsc.PackFormat.INTERLEAVED)
                        fbuf[r, pl.ds(32 * c, 16)] = a.astype(jnp.float32)
                        fbuf[r, pl.ds(32 * c + 16, 16)] = b.astype(jnp.float32)
                return carry
            lax.fori_loop(0, chunk // 4, quad, 0)

        def halfstep(k, j, gbuf, glsem, sbuf, isem, rbuf, gsem, fbuf, ssem):
            @pl.when(k > 0)
            def _():
                pltpu.make_async_copy(fbuf, acc.at[sbuf], ssem).wait()
            start_sidx(j, sbuf, isem)
            pltpu.make_async_copy(table.at[gbuf], rbuf, gsem).wait()
            @pl.when(j + 2 < nchunk)
            def _():
                start_gidx(j + 2, gbuf, glsem)
            convert(rbuf, fbuf)
            wait_sidx(j, sbuf, isem)
            pltpu.async_copy(fbuf, acc.at[sbuf], ssem, add=True)
            @pl.when(j + 2 < nchunk)
            def _():
                wait_gidx(j + 2, gbuf, glsem)
                pltpu.async_copy(table.at[gbuf], rbuf, gsem)

        pltpu.sync_copy(gidx.at[pl.ds(base0, chunk)], gb0)
        pltpu.sync_copy(gidx.at[pl.ds(base0 + chunk, chunk)], gb1)
        pltpu.async_copy(table.at[gb0], rb0, gsem0)
        pltpu.async_copy(table.at[gb1], rb1, gsem1)

        def body(k, carry):
            halfstep(k, 2 * k, gb0, glsem0, sb0, isem0, rb0, gsem0, rf0, ssem0)
            halfstep(k, 2 * k + 1, gb1, glsem1, sb1, isem1, rb1, gsem1,
                     rf1, ssem1)
            return carry

        npair = nchunk // 2
        lax.fori_loop(0, npair, body, 0)
        if nchunk % 2 == 1:
            halfstep(npair, nchunk - 1, gb0, glsem0, sb0, isem0, rb0, gsem0,
                     rf0, ssem0)
        pltpu.make_async_copy(rf0, acc.at[sb0], ssem0).wait()
        pltpu.make_async_copy(rf1, acc.at[sb1], ssem1).wait()
        plsc.subcore_barrier()
        pltpu.sync_copy(acc.at[pl.ds(sid * rz, rz)],
                        out.at[cid, pl.ds(sid * rz, rz)])

    return stage


def kernel(X, vertex, edges, degE, degV, W):
    N, D = X.shape
    OUT = W.shape[0]
    M = degE.shape[0]
    NNZ = vertex.shape[0]
    DA = OUT + 16

    u = _unpack_perm(OUT)
    uinv = np.argsort(u)
    Wp = W.T[:, uinv[uinv]]

    xp = _tc_linear_bf16(X, Wp)

    zA = jnp.zeros((M, DA), jnp.float32)
    pA = _make_sc_stage(M, NNZ, 80, DA, True)(xp, vertex, edges, zA)
    xe = _tc_mean_scale(pA, degE, OUT)

    zB = jnp.zeros((N, OUT), jnp.float32)
    pB = _make_sc_stage(N, NNZ, 80, OUT, False)(xe, edges, vertex, zB)
    return _tc_scale(pB, degV)


# speedup vs baseline: 1.0063x; 1.0063x over previous
import functools

import jax
import jax.numpy as jnp
import numpy as np
from jax import lax
from jax.experimental import pallas as pl
from jax.experimental.pallas import tpu as pltpu
from jax.experimental.pallas import tpu_sc as plsc

NC = 2
NS = 16
NW = NC * NS


def _unpack_perm(ncols):
    u = np.empty(ncols, np.int32)
    for k in range(ncols // 32):
        for i in range(16):
            u[32 * k + i] = 32 * k + 2 * i
            u[32 * k + 16 + i] = 32 * k + 2 * i + 1
    return u


def _tc_linear_bf16(X, Wt):
    N, D = X.shape
    OUT = Wt.shape[1]
    B = 1000

    def body(x_ref, w_ref, o_ref):
        acc = jnp.dot(x_ref[...], w_ref[...], preferred_element_type=jnp.float32)
        o_ref[...] = acc.astype(jnp.bfloat16)

    return pl.pallas_call(
        body,
        grid=(N // B,),
        in_specs=[pl.BlockSpec((B, D), lambda i: (i, 0)),
                  pl.BlockSpec((D, OUT), lambda i: (0, 0))],
        out_specs=pl.BlockSpec((B, OUT), lambda i: (i, 0)),
        out_shape=jax.ShapeDtypeStruct((N, OUT), jnp.bfloat16),
    )(X, Wt)


def _tc_mean_scale(partials, degE, OUT):
    _, M, DA = partials.shape
    B = 2000

    def body(p_ref, de_ref, o_ref):
        s = p_ref[0] + p_ref[1]
        cnt = s[:, OUT:OUT + 1]
        o_ref[...] = (s[:, :OUT] / jnp.maximum(cnt, 1.0)
                      * de_ref[...]).astype(jnp.bfloat16)

    return pl.pallas_call(
        body,
        grid=(M // B,),
        in_specs=[pl.BlockSpec((2, B, DA), lambda i: (0, i, 0)),
                  pl.BlockSpec((B, 1), lambda i: (i, 0))],
        out_specs=pl.BlockSpec((B, OUT), lambda i: (i, 0)),
        out_shape=jax.ShapeDtypeStruct((M, OUT), jnp.bfloat16),
    )(partials, degE)


def _tc_scale(partials, degV):
    _, N, OUT = partials.shape
    B = 2000

    def body(p_ref, dv_ref, o_ref):
        o_ref[...] = (p_ref[0] + p_ref[1]) * dv_ref[...]

    return pl.pallas_call(
        body,
        grid=(N // B,),
        in_specs=[pl.BlockSpec((2, B, OUT), lambda i: (0, i, 0)),
                  pl.BlockSpec((B, 1), lambda i: (i, 0))],
        out_specs=pl.BlockSpec((B, OUT), lambda i: (i, 0)),
        out_shape=jax.ShapeDtypeStruct((N, OUT), jnp.float32),
    )(partials, degV)


def _make_sc_stage(R, nnz, chunk, width, aug):
    D = 128
    P = nnz // NW
    nchunk = P // chunk
    rz = R // NS
    mesh = plsc.VectorSubcoreMesh(core_axis_name="c", subcore_axis_name="s")

    @functools.partial(
        pl.kernel,
        out_type=jax.ShapeDtypeStruct((NC, R, width), jnp.float32),
        mesh=mesh,
        compiler_params=pltpu.CompilerParams(use_tc_tiling_on_sc=False,
                                             needs_layout_passes=False),
        scratch_types=[
            pltpu.VMEM((chunk,), jnp.int32),
            pltpu.VMEM((chunk,), jnp.int32),
            pltpu.VMEM((chunk,), jnp.int32),
            pltpu.VMEM((chunk,), jnp.int32),
            pltpu.VMEM((chunk, D), jnp.bfloat16),
            pltpu.VMEM((chunk, D), jnp.bfloat16),
            pltpu.VMEM((chunk, width), jnp.float32),
            pltpu.VMEM((chunk, width), jnp.float32),
            pltpu.VMEM_SHARED((R, width), jnp.float32),
            pltpu.SemaphoreType.DMA,
            pltpu.SemaphoreType.DMA,
            pltpu.SemaphoreType.DMA,
            pltpu.SemaphoreType.DMA,
            pltpu.SemaphoreType.DMA,
            pltpu.SemaphoreType.DMA,
            pltpu.SemaphoreType.DMA,
            pltpu.SemaphoreType.DMA,
        ],
    )
    def stage(table, gidx, sidx, zeros, out, gb0, gb1, sb0, sb1, rb0, rb1,
              rf0, rf1, acc, gsem0, gsem1, glsem0, glsem1, isem0, isem1,
              ssem0, ssem1):
        cid = lax.axis_index("c")
        sid = lax.axis_index("s")
        wid = cid * NS + sid
        pltpu.sync_copy(zeros.at[pl.ds(sid * rz, rz)], acc.at[pl.ds(sid * rz, rz)])
        base0 = wid * P
        if aug:
            pad = jnp.where(lax.iota(jnp.int32, 16) == 0, 1.0, 0.0)
            for r in range(chunk):
                rf0[r, pl.ds(D, 16)] = pad
                rf1[r, pl.ds(D, 16)] = pad
        plsc.subcore_barrier()

        def start_gidx(j, gbuf, sem):
            pltpu.async_copy(gidx.at[pl.ds(base0 + j * chunk, chunk)],
                             gbuf, sem)

        def wait_gidx(j, gbuf, sem):
            pltpu.make_async_copy(gidx.at[pl.ds(base0 + j * chunk, chunk)],
                                  gbuf, sem).wait()

        def start_sidx(j, sbuf, sem):
            pltpu.async_copy(sidx.at[pl.ds(base0 + j * chunk, chunk)],
                             sbuf, sem)

        def wait_sidx(j, sbuf, sem):
            pltpu.make_async_copy(sidx.at[pl.ds(base0 + j * chunk, chunk)],
                                  sbuf, sem).wait()

        def convert(rbuf, fbuf):
            def group(q, carry):
                r0 = q * 8
                for dr in range(8):
                    r = r0 + dr
                    for c in range(D // 32):
                        v = rbuf[r, pl.ds(32 * c, 32)]
                        a, b = plsc.unpack(
                            v, format=plsc.PackFormat.INTERLEAVED)
                        fbuf[r, pl.ds(32 * c, 16)] = a.astype(jnp.float32)
                        fbuf[r, pl.ds(32 * c + 16, 16)] = b.astype(jnp.float32)
                return carry
            lax.fori_loop(0, chunk // 8, group, 0)

        def halfstep(k, j, gbuf, glsem, sbuf, isem, rbuf, gsem, fbuf, ssem):
            @pl.when(k > 0)
            def _():
                pltpu.make_async_copy(fbuf, acc.at[sbuf], ssem).wait()
            start_sidx(j, sbuf, isem)
            pltpu.make_async_copy(table.at[gbuf], rbuf, gsem).wait()
            @pl.when(j + 2 < nchunk)
            def _():
                start_gidx(j + 2, gbuf, glsem)
            convert(rbuf, fbuf)
            wait_sidx(j, sbuf, isem)
            pltpu.async_copy(fbuf, acc.at[sbuf], ssem, add=True)
            @pl.when(j + 2 < nchunk)
            def _():
                wait_gidx(j + 2, gbuf, glsem)
                pltpu.async_copy(table.at[gbuf], rbuf, gsem)

        pltpu.sync_copy(gidx.at[pl.ds(base0, chunk)], gb0)
        pltpu.sync_copy(gidx.at[pl.ds(base0 + chunk, chunk)], gb1)
        pltpu.async_copy(table.at[gb0], rb0, gsem0)
        pltpu.async_copy(table.at[gb1], rb1, gsem1)

        def body(k, carry):
            halfstep(k, 2 * k, gb0, glsem0, sb0, isem0, rb0, gsem0, rf0, ssem0)
            halfstep(k, 2 * k + 1, gb1, glsem1, sb1, isem1, rb1, gsem1,
                     rf1, ssem1)
            return carry

        npair = nchunk // 2
        lax.fori_loop(0, npair, body, 0)
        if nchunk % 2 == 1:
            halfstep(npair, nchunk - 1, gb0, glsem0, sb0, isem0, rb0, gsem0,
                     rf0, ssem0)
        pltpu.make_async_copy(rf0, acc.at[sb0], ssem0).wait()
        pltpu.make_async_copy(rf1, acc.at[sb1], ssem1).wait()
        plsc.subcore_barrier()
        pltpu.sync_copy(acc.at[pl.ds(sid * rz, rz)],
                        out.at[cid, pl.ds(sid * rz, rz)])

    return stage


def kernel(X, vertex, edges, degE, degV, W):
    N, D = X.shape
    OUT = W.shape[0]
    M = degE.shape[0]
    NNZ = vertex.shape[0]
    DA = OUT + 16

    u = _unpack_perm(OUT)
    uinv = np.argsort(u)
    Wp = W.T[:, uinv[uinv]]

    xp = _tc_linear_bf16(X, Wp)

    zA = jnp.zeros((M, DA), jnp.float32)
    pA = _make_sc_stage(M, NNZ, 80, DA, True)(xp, vertex, edges, zA)
    xe = _tc_mean_scale(pA, degE, OUT)

    zB = jnp.zeros((N, OUT), jnp.float32)
    pB = _make_sc_stage(N, NNZ, 80, OUT, False)(xe, edges, vertex, zB)
    return _tc_scale(pB, degV)


# speedup vs baseline: 1.0063x; 1.0000x over previous
import functools

import jax
import jax.numpy as jnp
import numpy as np
from jax import lax
from jax.experimental import pallas as pl
from jax.experimental.pallas import tpu as pltpu
from jax.experimental.pallas import tpu_sc as plsc

NC = 2
NS = 16
NW = NC * NS


def _unpack_perm(ncols):
    u = np.empty(ncols, np.int32)
    for k in range(ncols // 32):
        for i in range(16):
            u[32 * k + i] = 32 * k + 2 * i
            u[32 * k + 16 + i] = 32 * k + 2 * i + 1
    return u


def _tc_linear_bf16(X, Wt):
    N, D = X.shape
    OUT = Wt.shape[1]
    B = 1000

    def body(x_ref, w_ref, o_ref):
        acc = jnp.dot(x_ref[...], w_ref[...], preferred_element_type=jnp.float32)
        o_ref[...] = acc.astype(jnp.bfloat16)

    return pl.pallas_call(
        body,
        grid=(N // B,),
        in_specs=[pl.BlockSpec((B, D), lambda i: (i, 0)),
                  pl.BlockSpec((D, OUT), lambda i: (0, 0))],
        out_specs=pl.BlockSpec((B, OUT), lambda i: (i, 0)),
        out_shape=jax.ShapeDtypeStruct((N, OUT), jnp.bfloat16),
    )(X, Wt)


def _tc_mean_scale(partials, degE, OUT):
    _, M, DA = partials.shape
    B = 2000

    def body(p_ref, de_ref, o_ref):
        s = p_ref[0] + p_ref[1]
        cnt = s[:, OUT:OUT + 1]
        o_ref[...] = (s[:, :OUT] / jnp.maximum(cnt, 1.0)
                      * de_ref[...]).astype(jnp.bfloat16)

    return pl.pallas_call(
        body,
        grid=(M // B,),
        in_specs=[pl.BlockSpec((2, B, DA), lambda i: (0, i, 0)),
                  pl.BlockSpec((B, 1), lambda i: (i, 0))],
        out_specs=pl.BlockSpec((B, OUT), lambda i: (i, 0)),
        out_shape=jax.ShapeDtypeStruct((M, OUT), jnp.bfloat16),
    )(partials, degE)


def _tc_scale(partials, degV):
    _, N, OUT = partials.shape
    B = 2000

    def body(p_ref, dv_ref, o_ref):
        o_ref[...] = (p_ref[0] + p_ref[1]) * dv_ref[...]

    return pl.pallas_call(
        body,
        grid=(N // B,),
        in_specs=[pl.BlockSpec((2, B, OUT), lambda i: (0, i, 0)),
                  pl.BlockSpec((B, 1), lambda i: (i, 0))],
        out_specs=pl.BlockSpec((B, OUT), lambda i: (i, 0)),
        out_shape=jax.ShapeDtypeStruct((N, OUT), jnp.float32),
    )(partials, degV)


def _make_sc_stage(R, nnz, chunk, width, aug):
    D = 128
    P = nnz // NW
    nchunk = P // chunk
    rz = R // NS
    mesh = plsc.VectorSubcoreMesh(core_axis_name="c", subcore_axis_name="s")

    @functools.partial(
        pl.kernel,
        out_type=jax.ShapeDtypeStruct((NC, R, width), jnp.float32),
        mesh=mesh,
        compiler_params=pltpu.CompilerParams(use_tc_tiling_on_sc=False,
                                             needs_layout_passes=False),
        scratch_types=[
            pltpu.VMEM((chunk,), jnp.int32),
            pltpu.VMEM((chunk,), jnp.int32),
            pltpu.VMEM((chunk,), jnp.int32),
            pltpu.VMEM((chunk,), jnp.int32),
            pltpu.VMEM((chunk, D), jnp.bfloat16),
            pltpu.VMEM((chunk, D), jnp.bfloat16),
            pltpu.VMEM((chunk, width), jnp.float32),
            pltpu.VMEM((chunk, width), jnp.float32),
            pltpu.VMEM_SHARED((R, width), jnp.float32),
            pltpu.SemaphoreType.DMA,
            pltpu.SemaphoreType.DMA,
            pltpu.SemaphoreType.DMA,
            pltpu.SemaphoreType.DMA,
            pltpu.SemaphoreType.DMA,
            pltpu.SemaphoreType.DMA,
            pltpu.SemaphoreType.DMA,
            pltpu.SemaphoreType.DMA,
        ],
    )
    def stage(table, gidx, sidx, zeros, out, gb0, gb1, sb0, sb1, rb0, rb1,
              rf0, rf1, acc, gsem0, gsem1, glsem0, glsem1, isem0, isem1,
              ssem0, ssem1):
        cid = lax.axis_index("c")
        sid = lax.axis_index("s")
        wid = cid * NS + sid
        pltpu.sync_copy(zeros.at[pl.ds(sid * rz, rz)], acc.at[pl.ds(sid * rz, rz)])
        base0 = wid * P
        if aug:
            pad = jnp.where(lax.iota(jnp.int32, 16) == 0, 1.0, 0.0)
            for r in range(chunk):
                rf0[r, pl.ds(D, 16)] = pad
                rf1[r, pl.ds(D, 16)] = pad
        plsc.subcore_barrier()

        def start_gidx(j, gbuf, sem):
            pltpu.async_copy(gidx.at[pl.ds(base0 + j * chunk, chunk)],
                             gbuf, sem)

        def wait_gidx(j, gbuf, sem):
            pltpu.make_async_copy(gidx.at[pl.ds(base0 + j * chunk, chunk)],
                                  gbuf, sem).wait()

        def start_sidx(j, sbuf, sem):
            pltpu.async_copy(sidx.at[pl.ds(base0 + j * chunk, chunk)],
                             sbuf, sem)

        def wait_sidx(j, sbuf, sem):
            pltpu.make_async_copy(sidx.at[pl.ds(base0 + j * chunk, chunk)],
                                  sbuf, sem).wait()

        def convert(rbuf, fbuf):
            def group(q, carry):
                r0 = q * 8
                for dr in range(8):
                    r = r0 + dr
                    for c in range(D // 32):
                        v = rbuf[r, pl.ds(32 * c, 32)]
                        w = plsc.bitcast(v, jnp.int32)
                        lo = plsc.bitcast(w << 16, jnp.float32)
                        hi = plsc.bitcast(w & jnp.int32(-65536), jnp.float32)
                        fbuf[r, pl.ds(32 * c, 16)] = lo
                        fbuf[r, pl.ds(32 * c + 16, 16)] = hi
                return carry
            lax.fori_loop(0, chunk // 8, group, 0)

        def halfstep(k, j, gbuf, glsem, sbuf, isem, rbuf, gsem, fbuf, ssem):
            @pl.when(k > 0)
            def _():
                pltpu.make_async_copy(fbuf, acc.at[sbuf], ssem).wait()
            start_sidx(j, sbuf, isem)
            pltpu.make_async_copy(table.at[gbuf], rbuf, gsem).wait()
            @pl.when(j + 2 < nchunk)
            def _():
                start_gidx(j + 2, gbuf, glsem)
            convert(rbuf, fbuf)
            wait_sidx(j, sbuf, isem)
            pltpu.async_copy(fbuf, acc.at[sbuf], ssem, add=True)
            @pl.when(j + 2 < nchunk)
            def _():
                wait_gidx(j + 2, gbuf, glsem)
                pltpu.async_copy(table.at[gbuf], rbuf, gsem)

        pltpu.sync_copy(gidx.at[pl.ds(base0, chunk)], gb0)
        pltpu.sync_copy(gidx.at[pl.ds(base0 + chunk, chunk)], gb1)
        pltpu.async_copy(table.at[gb0], rb0, gsem0)
        pltpu.async_copy(table.at[gb1], rb1, gsem1)

        def body(k, carry):
            halfstep(k, 2 * k, gb0, glsem0, sb0, isem0, rb0, gsem0, rf0, ssem0)
            halfstep(k, 2 * k + 1, gb1, glsem1, sb1, isem1, rb1, gsem1,
                     rf1, ssem1)
            return carry

        npair = nchunk // 2
        lax.fori_loop(0, npair, body, 0)
        if nchunk % 2 == 1:
            halfstep(npair, nchunk - 1, gb0, glsem0, sb0, isem0, rb0, gsem0,
                     rf0, ssem0)
        pltpu.make_async_copy(rf0, acc.at[sb0], ssem0).wait()
        pltpu.make_async_copy(rf1, acc.at[sb1], ssem1).wait()
        plsc.subcore_barrier()
        pltpu.sync_copy(acc.at[pl.ds(sid * rz, rz)],
                        out.at[cid, pl.ds(sid * rz, rz)])

    return stage


def kernel(X, vertex, edges, degE, degV, W):
    N, D = X.shape
    OUT = W.shape[0]
    M = degE.shape[0]
    NNZ = vertex.shape[0]
    DA = OUT + 16

    u = _unpack_perm(OUT)
    uinv = np.argsort(u)
    Wp = W.T[:, uinv[uinv]]

    xp = _tc_linear_bf16(X, Wp)

    zA = jnp.zeros((M, DA), jnp.float32)
    pA = _make_sc_stage(M, NNZ, 80, DA, True)(xp, vertex, edges, zA)
    xe = _tc_mean_scale(pA, degE, OUT)

    zB = jnp.zeros((N, OUT), jnp.float32)
    pB = _make_sc_stage(N, NNZ, 80, OUT, False)(xe, edges, vertex, zB)
    return _tc_scale(pB, degV)


# speedup vs baseline: 1.6450x; 1.6346x over previous
import functools

import jax
import jax.numpy as jnp
import numpy as np
from jax import lax
from jax.experimental import pallas as pl
from jax.experimental.pallas import tpu as pltpu
from jax.experimental.pallas import tpu_sc as plsc

NC = 2
NS = 16
NW = NC * NS


def _unpack_perm(ncols):
    u = np.empty(ncols, np.int32)
    for k in range(ncols // 32):
        for i in range(16):
            u[32 * k + i] = 32 * k + 2 * i
            u[32 * k + 16 + i] = 32 * k + 2 * i + 1
    return u


def _tc_linear_bf16(X, Wt):
    N, D = X.shape
    OUT = Wt.shape[1]
    B = 1000

    def body(x_ref, w_ref, o_ref):
        acc = jnp.dot(x_ref[...], w_ref[...], preferred_element_type=jnp.float32)
        o_ref[...] = acc.astype(jnp.bfloat16)

    return pl.pallas_call(
        body,
        grid=(N // B,),
        in_specs=[pl.BlockSpec((B, D), lambda i: (i, 0)),
                  pl.BlockSpec((D, OUT), lambda i: (0, 0))],
        out_specs=pl.BlockSpec((B, OUT), lambda i: (i, 0)),
        out_shape=jax.ShapeDtypeStruct((N, OUT), jnp.bfloat16),
    )(X, Wt)


def _tc_mean_scale(partials, degE, OUT):
    _, M, DA = partials.shape
    B = 2000

    def body(p_ref, de_ref, o_ref):
        s = p_ref[0] + p_ref[1]
        cnt = s[:, OUT:OUT + 1]
        o_ref[...] = (s[:, :OUT] / jnp.maximum(cnt, 1.0)
                      * de_ref[...]).astype(jnp.bfloat16)

    return pl.pallas_call(
        body,
        grid=(M // B,),
        in_specs=[pl.BlockSpec((2, B, DA), lambda i: (0, i, 0)),
                  pl.BlockSpec((B, 1), lambda i: (i, 0))],
        out_specs=pl.BlockSpec((B, OUT), lambda i: (i, 0)),
        out_shape=jax.ShapeDtypeStruct((M, OUT), jnp.bfloat16),
    )(partials, degE)


def _tc_scale(partials, degV):
    _, N, OUT = partials.shape
    B = 2000

    def body(p_ref, dv_ref, o_ref):
        o_ref[...] = (p_ref[0] + p_ref[1]) * dv_ref[...]

    return pl.pallas_call(
        body,
        grid=(N // B,),
        in_specs=[pl.BlockSpec((2, B, OUT), lambda i: (0, i, 0)),
                  pl.BlockSpec((B, 1), lambda i: (i, 0))],
        out_specs=pl.BlockSpec((B, OUT), lambda i: (i, 0)),
        out_shape=jax.ShapeDtypeStruct((N, OUT), jnp.float32),
    )(partials, degV)


def _make_sc_stage(R, nnz, chunk, width, aug):
    D = 128
    P = nnz // NW
    nchunk = P // chunk
    rz = R // NS
    mesh = plsc.VectorSubcoreMesh(core_axis_name="c", subcore_axis_name="s")

    @functools.partial(
        pl.kernel,
        out_type=jax.ShapeDtypeStruct((NC, R, width), jnp.float32),
        mesh=mesh,
        compiler_params=pltpu.CompilerParams(use_tc_tiling_on_sc=False,
                                             needs_layout_passes=False),
        scratch_types=[
            pltpu.VMEM((chunk,), jnp.int32),
            pltpu.VMEM((chunk,), jnp.int32),
            pltpu.VMEM((chunk,), jnp.int32),
            pltpu.VMEM((chunk,), jnp.int32),
            pltpu.VMEM((chunk, D), jnp.bfloat16),
            pltpu.VMEM((chunk, D), jnp.bfloat16),
            pltpu.VMEM((chunk, width), jnp.float32),
            pltpu.VMEM((chunk, width), jnp.float32),
            pltpu.VMEM_SHARED((R, width), jnp.float32),
            pltpu.SemaphoreType.DMA,
            pltpu.SemaphoreType.DMA,
            pltpu.SemaphoreType.DMA,
            pltpu.SemaphoreType.DMA,
            pltpu.SemaphoreType.DMA,
            pltpu.SemaphoreType.DMA,
            pltpu.SemaphoreType.DMA,
            pltpu.SemaphoreType.DMA,
        ],
    )
    def stage(table, gidx, sidx, zeros, out, gb0, gb1, sb0, sb1, rb0, rb1,
              rf0, rf1, acc, gsem0, gsem1, glsem0, glsem1, isem0, isem1,
              ssem0, ssem1):
        cid = lax.axis_index("c")
        sid = lax.axis_index("s")
        wid = cid * NS + sid
        pltpu.sync_copy(zeros.at[pl.ds(sid * rz, rz)], acc.at[pl.ds(sid * rz, rz)])
        base0 = wid * P
        if aug:
            pad = jnp.where(lax.iota(jnp.int32, 16) == 0, 1.0, 0.0)
            for r in range(chunk):
                rf0[r, pl.ds(D, 16)] = pad
                rf1[r, pl.ds(D, 16)] = pad
        plsc.subcore_barrier()

        def start_gidx(j, gbuf, sem):
            pltpu.async_copy(gidx.at[pl.ds(base0 + j * chunk, chunk)],
                             gbuf, sem)

        def wait_gidx(j, gbuf, sem):
            pltpu.make_async_copy(gidx.at[pl.ds(base0 + j * chunk, chunk)],
                                  gbuf, sem).wait()

        def start_sidx(j, sbuf, sem):
            pltpu.async_copy(sidx.at[pl.ds(base0 + j * chunk, chunk)],
                             sbuf, sem)

        def wait_sidx(j, sbuf, sem):
            pltpu.make_async_copy(sidx.at[pl.ds(base0 + j * chunk, chunk)],
                                  sbuf, sem).wait()

        def convert(rbuf, fbuf):
            @plsc.parallel_loop(0, chunk, step=1)
            def _(r):
                for c in range(D // 32):
                    v = rbuf[r, pl.ds(32 * c, 32)]
                    w = plsc.bitcast(v, jnp.int32)
                    lo = plsc.bitcast(w << 16, jnp.float32)
                    hi = plsc.bitcast(w & jnp.int32(-65536), jnp.float32)
                    fbuf[r, pl.ds(32 * c, 16)] = lo
                    fbuf[r, pl.ds(32 * c + 16, 16)] = hi

        def halfstep(k, j, gbuf, glsem, sbuf, isem, rbuf, gsem, fbuf, ssem):
            @pl.when(k > 0)
            def _():
                pltpu.make_async_copy(fbuf, acc.at[sbuf], ssem).wait()
            start_sidx(j, sbuf, isem)
            pltpu.make_async_copy(table.at[gbuf], rbuf, gsem).wait()
            @pl.when(j + 2 < nchunk)
            def _():
                start_gidx(j + 2, gbuf, glsem)
            convert(rbuf, fbuf)
            wait_sidx(j, sbuf, isem)
            pltpu.async_copy(fbuf, acc.at[sbuf], ssem, add=True)
            @pl.when(j + 2 < nchunk)
            def _():
                wait_gidx(j + 2, gbuf, glsem)
                pltpu.async_copy(table.at[gbuf], rbuf, gsem)

        pltpu.sync_copy(gidx.at[pl.ds(base0, chunk)], gb0)
        pltpu.sync_copy(gidx.at[pl.ds(base0 + chunk, chunk)], gb1)
        pltpu.async_copy(table.at[gb0], rb0, gsem0)
        pltpu.async_copy(table.at[gb1], rb1, gsem1)

        def body(k, carry):
            halfstep(k, 2 * k, gb0, glsem0, sb0, isem0, rb0, gsem0, rf0, ssem0)
            halfstep(k, 2 * k + 1, gb1, glsem1, sb1, isem1, rb1, gsem1,
                     rf1, ssem1)
            return carry

        npair = nchunk // 2
        lax.fori_loop(0, npair, body, 0)
        if nchunk % 2 == 1:
            halfstep(npair, nchunk - 1, gb0, glsem0, sb0, isem0, rb0, gsem0,
                     rf0, ssem0)
        pltpu.make_async_copy(rf0, acc.at[sb0], ssem0).wait()
        pltpu.make_async_copy(rf1, acc.at[sb1], ssem1).wait()
        plsc.subcore_barrier()
        pltpu.sync_copy(acc.at[pl.ds(sid * rz, rz)],
                        out.at[cid, pl.ds(sid * rz, rz)])

    return stage


def kernel(X, vertex, edges, degE, degV, W):
    N, D = X.shape
    OUT = W.shape[0]
    M = degE.shape[0]
    NNZ = vertex.shape[0]
    DA = OUT + 16

    u = _unpack_perm(OUT)
    uinv = np.argsort(u)
    Wp = W.T[:, uinv[uinv]]

    xp = _tc_linear_bf16(X, Wp)

    zA = jnp.zeros((M, DA), jnp.float32)
    pA = _make_sc_stage(M, NNZ, 80, DA, True)(xp, vertex, edges, zA)
    xe = _tc_mean_scale(pA, degE, OUT)

    zB = jnp.zeros((N, OUT), jnp.float32)
    pB = _make_sc_stage(N, NNZ, 80, OUT, False)(xe, edges, vertex, zB)
    return _tc_scale(pB, degV)
